# baseline (speedup 1.0000x reference)
import functools

import jax
import jax.numpy as jnp
from jax import lax
from jax.experimental import pallas as pl
from jax.experimental.pallas import tpu as pltpu
from jax.experimental.pallas import tpu_sc as plsc

N = 10000
E = 160000
IN = 128
HID = 256
OUT = 128
DE = 16
NE = 8

BLK = 256
NG = (N + BLK - 1) // BLK

EB = 128
NC = 2
NS = 16
NW = NC * NS
JT = 40
EPAD = NW * JT * EB
WCHUNK = 632
NPAD = NS * WCHUNK


@functools.lru_cache(maxsize=None)
def _make_scpass(gather):
    mesh = plsc.VectorSubcoreMesh(core_axis_name="c", subcore_axis_name="s")

    @functools.partial(
        pl.kernel,
        mesh=mesh,
        out_type=jax.ShapeDtypeStruct((NC, NPAD, 128), jnp.float32),
        scratch_types=[
            pltpu.VMEM((EB,), jnp.int32),
            pltpu.VMEM((EB,), jnp.int32),
            pltpu.VMEM((EB,), jnp.int32),
            pltpu.VMEM((EB,), jnp.int32),
            pltpu.VMEM((EB,), jnp.int32),
            pltpu.VMEM((EB,), jnp.int32),
            pltpu.VMEM((EB, 128), jnp.float32),
            pltpu.VMEM((EB, 128), jnp.float32),
            pltpu.VMEM((EB, 128), jnp.float32),
            pltpu.VMEM_SHARED((NPAD, 128), jnp.float32),
            pltpu.SemaphoreType.DMA,
            pltpu.SemaphoreType.DMA,
            pltpu.SemaphoreType.DMA,
            pltpu.SemaphoreType.DMA,
            pltpu.SemaphoreType.DMA,
            pltpu.SemaphoreType.DMA,
        ],
    )
    def k(vals, src3d, dst3d, zrows, out,
          sa0, sa1, sa2, da0, da1, da2, b0, b1, b2, acc,
          g0, g1, g2, i0, i1, i2):
        cid = lax.axis_index("c")
        sid = lax.axis_index("s")
        wid = sid * NC + cid
        pltpu.sync_copy(zrows, acc.at[pl.ds(sid * WCHUNK, WCHUNK)])
        plsc.subcore_barrier()
        base = wid * JT * EB
        sidxb = (sa0, sa1, sa2)
        didxb = (da0, da1, da2)
        bufs = (b0, b1, b2)
        gsem = (g0, g1, g2)
        isem = (i0, i1, i2)

        def run():
            def idx_start(j, p):
                if gather:
                    pltpu.async_copy(src3d.at[wid, j], sidxb[p], isem[p])
                pltpu.async_copy(dst3d.at[wid, j], didxb[p], isem[p])

            def idx_wait(j, p):
                if gather:
                    pltpu.make_async_copy(
                        src3d.at[wid, j], sidxb[p], isem[p]).wait()
                pltpu.make_async_copy(
                    dst3d.at[wid, j], didxb[p], isem[p]).wait()

            def gsrc(j, p):
                if gather:
                    return vals.at[sidxb[p]]
                off = jnp.minimum(base + j * EB, E - EB)
                return vals.at[pl.ds(off, EB)]

            def gat_start(j, p):
                pltpu.async_copy(gsrc(j, p), bufs[p], gsem[p])

            def gat_drain(j, p):
                pltpu.make_async_copy(gsrc(j, p), bufs[p], gsem[p]).wait()

            idx_start(0, 0)
            idx_start(1, 1)
            idx_start(2, 2)
            idx_wait(0, 0)
            gat_start(0, 0)
            idx_wait(1, 1)
            gat_start(1, 1)

            def body(t, carry):
                for r in (0, 1, 2):
                    j = 3 * t + r

                    @pl.when(j < JT)
                    def _(j=j, r=r):
                        gat_drain(j, r)
                        pltpu.sync_copy(bufs[r], acc.at[didxb[r]], add=True)

                        @pl.when(j + 2 < JT)
                        def _(j=j, r=r):
                            p2 = (r + 2) % 3
                            idx_wait(j + 2, p2)
                            gat_start(j + 2, p2)

                        @pl.when(j + 3 < JT)
                        def _(j=j, r=r):
                            idx_start(j + 3, r)

                return carry

            lax.fori_loop(0, (JT + 2) // 3, body, 0)

        run()

        plsc.subcore_barrier()
        pltpu.sync_copy(acc.at[pl.ds(sid * WCHUNK, WCHUNK)],
                        out.at[cid, pl.ds(sid * WCHUNK, WCHUNK)])

    return k


def _stats_body(x_ref, o_ref):
    i = pl.program_id(0)
    rows = lax.broadcasted_iota(jnp.int32, (BLK, 1), 0) + i * BLK
    xb = jnp.where(rows < N, x_ref[...], 0.0)
    contrib = jnp.concatenate(
        [jnp.sum(xb, axis=0, keepdims=True),
         jnp.sum(xb * xb, axis=0, keepdims=True)], axis=0)

    @pl.when(i == 0)
    def _():
        o_ref[...] = jnp.zeros_like(o_ref)

    o_ref[...] += contrib


def _stats(x):
    D = x.shape[1]
    return pl.pallas_call(
        _stats_body,
        grid=(NG,),
        in_specs=[pl.BlockSpec((BLK, D), lambda i: (i, 0))],
        out_specs=pl.BlockSpec((2, D), lambda i: (0, 0)),
        out_shape=jax.ShapeDtypeStruct((2, D), jnp.float32),
    )(x)


def _expert_body(Din, Dout, nh, *refs):
    (x_ref, *agg_refs) = refs[:1 + nh]
    (agge_ref, stats_ref, g_ref, b_ref, wg_ref, wall_ref, bbf_ref,
     out_ref, topi_ref, imp_ref, loss_ref, st2_ref) = refs[1 + nh:]
    i = pl.program_id(0)
    rows = lax.broadcasted_iota(jnp.int32, (BLK, 1), 0) + i * BLK
    valid = rows < N

    m = stats_ref[0:1, :] / N
    var = stats_ref[1:2, :] / N - m * m
    A = g_ref[...] / jnp.sqrt(var + 1e-5)
    B = b_ref[...] - m * A

    xb = jnp.where(valid, x_ref[...], 0.0)
    h = xb * A + B

    if nh == 1:
        aggr = agg_refs[0][0] + agg_refs[0][1]
    else:
        aggr = jnp.concatenate([agg_refs[0][0] + agg_refs[0][1],
                                agg_refs[1][0] + agg_refs[1][1]], axis=1)
    aggr = jnp.where(valid, aggr, 0.0)
    e32 = jnp.where(valid, agge_ref[0] + agge_ref[1], 0.0)
    agge = e32[:, :DE]
    deg = e32[:, DE:DE + 1]
    aggx = aggr * A + deg * B

    logits = jnp.dot(h, wg_ref[...], preferred_element_type=jnp.float32)
    iota8 = lax.broadcasted_iota(jnp.int32, (BLK, NE), 1)
    v1 = jnp.max(logits, axis=1, keepdims=True)
    i1 = jnp.min(jnp.where(logits == v1, iota8, NE), axis=1, keepdims=True)
    masked = jnp.where(iota8 == i1, -jnp.inf, logits)
    v2 = jnp.max(masked, axis=1, keepdims=True)
    i2 = jnp.min(jnp.where(masked == v2, iota8, NE), axis=1, keepdims=True)
    gv1 = 1.0 / (1.0 + jnp.exp(v2 - v1))
    gv2 = 1.0 - gv1
    topi_ref[...] = jnp.concatenate([i1, i2], axis=1)
    gd = jnp.where(iota8 == i1, gv1, 0.0) + jnp.where(iota8 == i2, gv2, 0.0)
    gd = jnp.where(valid, gd, 0.0)

    @pl.when(i == 0)
    def _():
        imp_ref[...] = jnp.zeros_like(imp_ref)
        st2_ref[...] = jnp.zeros_like(st2_ref)

    imp_ref[...] += jnp.sum(gd, axis=0, keepdims=True)

    z = jnp.concatenate([h, aggx, agge], axis=1)
    big = jnp.dot(z, wall_ref[...], preferred_element_type=jnp.float32)
    big = jnp.maximum(big + bbf_ref[...], 0.0)
    big = big.astype(jnp.bfloat16).astype(jnp.float32)
    gdt = gd.astype(jnp.bfloat16).astype(jnp.float32)
    acc = jnp.zeros((BLK, Dout), jnp.float32)
    for e in range(NE):
        acc = acc + gdt[:, e:e + 1] * big[:, e * Dout:(e + 1) * Dout]
    out_ref[...] = acc

    st2_ref[...] += jnp.concatenate(
        [jnp.sum(acc, axis=0, keepdims=True),
         jnp.sum(acc * acc, axis=0, keepdims=True)], axis=0)

    @pl.when(i == NG - 1)
    def _():
        imp = imp_ref[...]
        mi = jnp.sum(imp) / NE
        vi = jnp.sum((imp - mi) ** 2) / NE
        loss_ref[...] = (vi / (mi * mi + 1e-10)).reshape(1, 1)


def _expert_layer(x, aggs, agge, stats, g, b, Wg, Wall, bbf, Din, Dout):
    Z = 2 * Din + DE
    nh = len(aggs)
    body = functools.partial(_expert_body, Din, Dout, nh)

    def _part_spec():
        return pl.BlockSpec((NC, BLK, 128), lambda i: (0, i, 0))

    return pl.pallas_call(
        body,
        grid=(NG,),
        in_specs=[
            pl.BlockSpec((BLK, Din), lambda i: (i, 0)),
            *[_part_spec() for _ in aggs],
            _part_spec(),
            pl.BlockSpec((2, Din), lambda i: (0, 0)),
            pl.BlockSpec((1, Din), lambda i: (0, 0)),
            pl.BlockSpec((1, Din), lambda i: (0, 0)),
            pl.BlockSpec((Din, NE), lambda i: (0, 0)),
            pl.BlockSpec((Z, NE * Dout), lambda i: (0, 0)),
            pl.BlockSpec((1, NE * Dout), lambda i: (0, 0)),
        ],
        out_specs=[
            pl.BlockSpec((BLK, Dout), lambda i: (i, 0)),
            pl.BlockSpec((BLK, 2), lambda i: (i, 0)),
            pl.BlockSpec((1, NE), lambda i: (0, 0)),
            pl.BlockSpec((1, 1), lambda i: (0, 0)),
            pl.BlockSpec((2, Dout), lambda i: (0, 0)),
        ],
        out_shape=[
            jax.ShapeDtypeStruct((N, Dout), jnp.float32),
            jax.ShapeDtypeStruct((N, 2), jnp.int32),
            jax.ShapeDtypeStruct((1, NE), jnp.float32),
            jax.ShapeDtypeStruct((1, 1), jnp.float32),
            jax.ShapeDtypeStruct((2, Dout), jnp.float32),
        ],
    )(x, *aggs, agge, stats, g, b, Wg, Wall, bbf)


def _pack_weights(Ws, Wm, We, bb):
    W8 = jnp.concatenate([Ws, Wm, We], axis=1)
    Z = W8.shape[1]
    Dout = W8.shape[2]
    Wall = jnp.transpose(W8, (1, 0, 2)).reshape(Z, NE * Dout)
    return Wall, bb.reshape(1, NE * Dout)


def kernel(x, edge_index, edge_attr, g1, b1, g2, b2,
           Wg1, Ws1, Wm1, We1, bb1, Wg2, Ws2, Wm2, We2, bb2):
    padn = EPAD - E
    srcp = jnp.concatenate([edge_index[0], jnp.zeros((padn,), jnp.int32)])
    dstp = jnp.concatenate([edge_index[1], jnp.full((padn,), N, jnp.int32)])
    src3d = srcp.reshape(NW, JT, EB)
    dst3d = dstp.reshape(NW, JT, EB)
    ea128 = jnp.concatenate(
        [edge_attr, jnp.ones((E, 1), jnp.float32),
         jnp.zeros((E, 128 - DE - 1), jnp.float32)], axis=1)
    z128 = jnp.zeros((WCHUNK, 128), jnp.float32)

    aggE = _make_scpass(False)(ea128, src3d, dst3d, z128)
    aggX = _make_scpass(True)(x, src3d, dst3d, z128)
    stats1 = _stats(x)

    Wall1, bbf1 = _pack_weights(Ws1, Wm1, We1, bb1)
    h2pre, e1, imp1, l1, st2 = _expert_layer(
        x, [aggX], aggE, stats1,
        g1.reshape(1, IN), b1.reshape(1, IN), Wg1, Wall1, bbf1, IN, HID)
    del imp1

    aggB0 = _make_scpass(True)(h2pre[:, :128], src3d, dst3d, z128)
    aggB1 = _make_scpass(True)(h2pre[:, 128:], src3d, dst3d, z128)

    Wall2, bbf2 = _pack_weights(Ws2, Wm2, We2, bb2)
    out, e2, imp2, l2, _ = _expert_layer(
        h2pre, [aggB0, aggB1], aggE, st2,
        g2.reshape(1, HID), b2.reshape(1, HID), Wg2, Wall2, bbf2, HID, OUT)
    del imp2

    losses = jnp.stack([l1[0, 0], l2[0, 0]])
    return out, e1, e2, losses


# speedup vs baseline: 1.3848x; 1.3848x over previous
import functools

import jax
import jax.numpy as jnp
from jax import lax
from jax.experimental import pallas as pl
from jax.experimental.pallas import tpu as pltpu
from jax.experimental.pallas import tpu_sc as plsc

N = 10000
E = 160000
IN = 128
HID = 256
OUT = 128
DE = 16
NE = 8

BLK = 256
NG = (N + BLK - 1) // BLK

EB = 128
NC = 2
NS = 16
NW = NC * NS
NBLK = E // EB
JMAX = (NBLK + NW - 1) // NW
WCHUNK = 632
NPAD = NS * WCHUNK


@functools.lru_cache(maxsize=None)
def _make_scpass(gather):
    mesh = plsc.VectorSubcoreMesh(core_axis_name="c", subcore_axis_name="s")

    @functools.partial(
        pl.kernel,
        mesh=mesh,
        out_type=jax.ShapeDtypeStruct((NC, NPAD, 128), jnp.float32),
        scratch_types=[
            pltpu.VMEM((EB,), jnp.int32),
            pltpu.VMEM((EB,), jnp.int32),
            pltpu.VMEM((EB, 128), jnp.float32),
            pltpu.VMEM_SHARED((NPAD, 128), jnp.float32),
            pltpu.SemaphoreType.DMA,
        ],
    )
    def k(vals, src3d, dst3d, zrows, out, sidx, didx, buf, acc, sem):
        cid = lax.axis_index("c")
        sid = lax.axis_index("s")
        wid = sid * NC + cid
        pltpu.sync_copy(zrows, acc.at[pl.ds(sid * WCHUNK, WCHUNK)])
        plsc.subcore_barrier()

        def body(j, carry):
            bid = wid + NW * j

            @pl.when(bid < NBLK)
            def _():
                pltpu.sync_copy(dst3d.at[bid, 0], didx)
                if gather:
                    pltpu.sync_copy(src3d.at[bid, 0], sidx)
                    pltpu.async_copy(vals.at[sidx], buf, sem).wait()
                else:
                    pltpu.sync_copy(vals.at[pl.ds(bid * EB, EB)], buf)
                pltpu.sync_copy(buf, acc.at[didx], add=True)

            return carry

        lax.fori_loop(0, JMAX, body, 0)

        plsc.subcore_barrier()
        pltpu.sync_copy(acc.at[pl.ds(sid * WCHUNK, WCHUNK)],
                        out.at[cid, pl.ds(sid * WCHUNK, WCHUNK)])

    return k


def _stats_body(x_ref, o_ref):
    i = pl.program_id(0)
    rows = lax.broadcasted_iota(jnp.int32, (BLK, 1), 0) + i * BLK
    xb = jnp.where(rows < N, x_ref[...], 0.0)
    contrib = jnp.concatenate(
        [jnp.sum(xb, axis=0, keepdims=True),
         jnp.sum(xb * xb, axis=0, keepdims=True)], axis=0)

    @pl.when(i == 0)
    def _():
        o_ref[...] = jnp.zeros_like(o_ref)

    o_ref[...] += contrib


def _stats(x):
    D = x.shape[1]
    return pl.pallas_call(
        _stats_body,
        grid=(NG,),
        in_specs=[pl.BlockSpec((BLK, D), lambda i: (i, 0))],
        out_specs=pl.BlockSpec((2, D), lambda i: (0, 0)),
        out_shape=jax.ShapeDtypeStruct((2, D), jnp.float32),
    )(x)


def _expert_body(Din, Dout, nh, *refs):
    (x_ref, *agg_refs) = refs[:1 + nh]
    (agge_ref, stats_ref, g_ref, b_ref, wg_ref, wall_ref, bbf_ref,
     out_ref, topi_ref, imp_ref, loss_ref, st2_ref) = refs[1 + nh:]
    i = pl.program_id(0)
    rows = lax.broadcasted_iota(jnp.int32, (BLK, 1), 0) + i * BLK
    valid = rows < N

    m = stats_ref[0:1, :] / N
    var = stats_ref[1:2, :] / N - m * m
    A = g_ref[...] / jnp.sqrt(var + 1e-5)
    B = b_ref[...] - m * A

    xb = jnp.where(valid, x_ref[...], 0.0)
    h = xb * A + B

    if nh == 1:
        aggr = agg_refs[0][0] + agg_refs[0][1]
    else:
        aggr = jnp.concatenate([agg_refs[0][0] + agg_refs[0][1],
                                agg_refs[1][0] + agg_refs[1][1]], axis=1)
    aggr = jnp.where(valid, aggr, 0.0)
    e32 = jnp.where(valid, agge_ref[0] + agge_ref[1], 0.0)
    agge = e32[:, :DE]
    deg = e32[:, DE:DE + 1]
    aggx = aggr * A + deg * B

    logits = jnp.dot(h, wg_ref[...], preferred_element_type=jnp.float32)
    iota8 = lax.broadcasted_iota(jnp.int32, (BLK, NE), 1)
    v1 = jnp.max(logits, axis=1, keepdims=True)
    i1 = jnp.min(jnp.where(logits == v1, iota8, NE), axis=1, keepdims=True)
    masked = jnp.where(iota8 == i1, -jnp.inf, logits)
    v2 = jnp.max(masked, axis=1, keepdims=True)
    i2 = jnp.min(jnp.where(masked == v2, iota8, NE), axis=1, keepdims=True)
    gv1 = 1.0 / (1.0 + jnp.exp(v2 - v1))
    gv2 = 1.0 - gv1
    topi_ref[...] = jnp.concatenate([i1, i2], axis=1)
    gd = jnp.where(iota8 == i1, gv1, 0.0) + jnp.where(iota8 == i2, gv2, 0.0)
    gd = jnp.where(valid, gd, 0.0)

    @pl.when(i == 0)
    def _():
        imp_ref[...] = jnp.zeros_like(imp_ref)
        st2_ref[...] = jnp.zeros_like(st2_ref)

    imp_ref[...] += jnp.sum(gd, axis=0, keepdims=True)

    z = jnp.concatenate([h, aggx, agge], axis=1)
    big = jnp.dot(z, wall_ref[...], preferred_element_type=jnp.float32)
    big = jnp.maximum(big + bbf_ref[...], 0.0)
    big = big.astype(jnp.bfloat16).astype(jnp.float32)
    gdt = gd.astype(jnp.bfloat16).astype(jnp.float32)
    acc = jnp.zeros((BLK, Dout), jnp.float32)
    for e in range(NE):
        acc = acc + gdt[:, e:e + 1] * big[:, e * Dout:(e + 1) * Dout]
    out_ref[...] = acc

    st2_ref[...] += jnp.concatenate(
        [jnp.sum(acc, axis=0, keepdims=True),
         jnp.sum(acc * acc, axis=0, keepdims=True)], axis=0)

    @pl.when(i == NG - 1)
    def _():
        imp = imp_ref[...]
        mi = jnp.sum(imp) / NE
        vi = jnp.sum((imp - mi) ** 2) / NE
        loss_ref[...] = (vi / (mi * mi + 1e-10)).reshape(1, 1)


def _expert_layer(x, aggs, agge, stats, g, b, Wg, Wall, bbf, Din, Dout):
    Z = 2 * Din + DE
    nh = len(aggs)
    body = functools.partial(_expert_body, Din, Dout, nh)

    def _part_spec():
        return pl.BlockSpec((NC, BLK, 128), lambda i: (0, i, 0))

    return pl.pallas_call(
        body,
        grid=(NG,),
        in_specs=[
            pl.BlockSpec((BLK, Din), lambda i: (i, 0)),
            *[_part_spec() for _ in aggs],
            _part_spec(),
            pl.BlockSpec((2, Din), lambda i: (0, 0)),
            pl.BlockSpec((1, Din), lambda i: (0, 0)),
            pl.BlockSpec((1, Din), lambda i: (0, 0)),
            pl.BlockSpec((Din, NE), lambda i: (0, 0)),
            pl.BlockSpec((Z, NE * Dout), lambda i: (0, 0)),
            pl.BlockSpec((1, NE * Dout), lambda i: (0, 0)),
        ],
        out_specs=[
            pl.BlockSpec((BLK, Dout), lambda i: (i, 0)),
            pl.BlockSpec((BLK, 2), lambda i: (i, 0)),
            pl.BlockSpec((1, NE), lambda i: (0, 0)),
            pl.BlockSpec((1, 1), lambda i: (0, 0)),
            pl.BlockSpec((2, Dout), lambda i: (0, 0)),
        ],
        out_shape=[
            jax.ShapeDtypeStruct((N, Dout), jnp.float32),
            jax.ShapeDtypeStruct((N, 2), jnp.int32),
            jax.ShapeDtypeStruct((1, NE), jnp.float32),
            jax.ShapeDtypeStruct((1, 1), jnp.float32),
            jax.ShapeDtypeStruct((2, Dout), jnp.float32),
        ],
    )(x, *aggs, agge, stats, g, b, Wg, Wall, bbf)


def _pack_weights(Ws, Wm, We, bb):
    W8 = jnp.concatenate([Ws, Wm, We], axis=1)
    Z = W8.shape[1]
    Dout = W8.shape[2]
    Wall = jnp.transpose(W8, (1, 0, 2)).reshape(Z, NE * Dout)
    return Wall, bb.reshape(1, NE * Dout)


def kernel(x, edge_index, edge_attr, g1, b1, g2, b2,
           Wg1, Ws1, Wm1, We1, bb1, Wg2, Ws2, Wm2, We2, bb2):
    src3d = edge_index[0].reshape(NBLK, 1, EB)
    dst3d = edge_index[1].reshape(NBLK, 1, EB)
    ea128 = jnp.concatenate(
        [edge_attr, jnp.ones((E, 1), jnp.float32),
         jnp.zeros((E, 128 - DE - 1), jnp.float32)], axis=1)
    z128 = jnp.zeros((WCHUNK, 128), jnp.float32)

    aggE = _make_scpass(False)(ea128, src3d, dst3d, z128)
    aggX = _make_scpass(True)(x, src3d, dst3d, z128)
    stats1 = _stats(x)

    Wall1, bbf1 = _pack_weights(Ws1, Wm1, We1, bb1)
    h2pre, e1, imp1, l1, st2 = _expert_layer(
        x, [aggX], aggE, stats1,
        g1.reshape(1, IN), b1.reshape(1, IN), Wg1, Wall1, bbf1, IN, HID)
    del imp1

    aggB0 = _make_scpass(True)(h2pre[:, :128], src3d, dst3d, z128)
    aggB1 = _make_scpass(True)(h2pre[:, 128:], src3d, dst3d, z128)

    Wall2, bbf2 = _pack_weights(Ws2, Wm2, We2, bb2)
    out, e2, imp2, l2, _ = _expert_layer(
        h2pre, [aggB0, aggB1], aggE, st2,
        g2.reshape(1, HID), b2.reshape(1, HID), Wg2, Wall2, bbf2, HID, OUT)
    del imp2

    losses = jnp.stack([l1[0, 0], l2[0, 0]])
    return out, e1, e2, losses


# speedup vs baseline: 1.4012x; 1.0118x over previous
import functools

import jax
import jax.numpy as jnp
from jax import lax
from jax.experimental import pallas as pl
from jax.experimental.pallas import tpu as pltpu
from jax.experimental.pallas import tpu_sc as plsc

N = 10000
E = 160000
IN = 128
HID = 256
OUT = 128
DE = 16
NE = 8

BLK = 256
NG = (N + BLK - 1) // BLK

EB = 128
NC = 2
NS = 16
NW = NC * NS
NBLK = E // EB
JMAX = (NBLK + NW - 1) // NW
WCHUNK = 632
NPAD = NS * WCHUNK


@functools.lru_cache(maxsize=None)
def _make_scpass(gather):
    mesh = plsc.VectorSubcoreMesh(core_axis_name="c", subcore_axis_name="s")

    @functools.partial(
        pl.kernel,
        mesh=mesh,
        out_type=jax.ShapeDtypeStruct((NC, NPAD, 128), jnp.float32),
        scratch_types=[
            pltpu.VMEM((EB,), jnp.int32),
            pltpu.VMEM((EB,), jnp.int32),
            pltpu.VMEM((EB, 128), jnp.float32),
            pltpu.VMEM_SHARED((NPAD, 128), jnp.float32),
            pltpu.SemaphoreType.DMA,
        ],
    )
    def k(vals, src3d, dst3d, zrows, out, sidx, didx, buf, acc, sem):
        cid = lax.axis_index("c")
        sid = lax.axis_index("s")
        wid = sid * NC + cid
        pltpu.sync_copy(zrows, acc.at[pl.ds(sid * WCHUNK, WCHUNK)])
        plsc.subcore_barrier()

        def body(j, carry):
            bid = wid + NW * j

            @pl.when(bid < NBLK)
            def _():
                pltpu.sync_copy(dst3d.at[bid, 0], didx)
                if gather:
                    pltpu.sync_copy(src3d.at[bid, 0], sidx)
                    pltpu.async_copy(vals.at[sidx], buf, sem).wait()
                else:
                    pltpu.sync_copy(vals.at[pl.ds(bid * EB, EB)], buf)
                pltpu.sync_copy(buf, acc.at[didx], add=True)

            return carry

        lax.fori_loop(0, JMAX, body, 0)

        plsc.subcore_barrier()
        pltpu.sync_copy(acc.at[pl.ds(sid * WCHUNK, WCHUNK)],
                        out.at[cid, pl.ds(sid * WCHUNK, WCHUNK)])

    return k


def _stats_body(x_ref, o_ref):
    i = pl.program_id(0)
    rows = lax.broadcasted_iota(jnp.int32, (BLK, 1), 0) + i * BLK
    xb = jnp.where(rows < N, x_ref[...], 0.0)
    contrib = jnp.concatenate(
        [jnp.sum(xb, axis=0, keepdims=True),
         jnp.sum(xb * xb, axis=0, keepdims=True)], axis=0)

    @pl.when(i == 0)
    def _():
        o_ref[...] = jnp.zeros_like(o_ref)

    o_ref[...] += contrib


def _stats(x):
    D = x.shape[1]
    return pl.pallas_call(
        _stats_body,
        grid=(NG,),
        in_specs=[pl.BlockSpec((BLK, D), lambda i: (i, 0))],
        out_specs=pl.BlockSpec((2, D), lambda i: (0, 0)),
        out_shape=jax.ShapeDtypeStruct((2, D), jnp.float32),
    )(x)


def _expert_body(Din, Dout, nx, nh, no, *refs):
    x_refs = refs[:nx]
    agg_refs = refs[nx:nx + nh]
    (agge_ref, stats_ref, g_ref, b_ref, wg_ref, wall_ref, bbf_ref) = \
        refs[nx + nh:nx + nh + 7]
    out_refs = refs[nx + nh + 7:nx + nh + 7 + no]
    (topi_ref, imp_ref, loss_ref, st2_ref) = refs[nx + nh + 7 + no:]
    i = pl.program_id(0)
    rows = lax.broadcasted_iota(jnp.int32, (BLK, 1), 0) + i * BLK
    valid = rows < N

    m = stats_ref[0:1, :] / N
    var = stats_ref[1:2, :] / N - m * m
    A = g_ref[...] / jnp.sqrt(var + 1e-5)
    B = b_ref[...] - m * A

    if nx == 1:
        xraw = x_refs[0][...]
    else:
        xraw = jnp.concatenate([r[...] for r in x_refs], axis=1)
    xb = jnp.where(valid, xraw, 0.0)
    h = xb * A + B

    if nh == 1:
        aggr = agg_refs[0][0] + agg_refs[0][1]
    else:
        aggr = jnp.concatenate([agg_refs[0][0] + agg_refs[0][1],
                                agg_refs[1][0] + agg_refs[1][1]], axis=1)
    aggr = jnp.where(valid, aggr, 0.0)
    e32 = jnp.where(valid, agge_ref[0] + agge_ref[1], 0.0)
    agge = e32[:, :DE]
    deg = e32[:, DE:DE + 1]
    aggx = aggr * A + deg * B

    logits = jnp.dot(h, wg_ref[...], preferred_element_type=jnp.float32)
    iota8 = lax.broadcasted_iota(jnp.int32, (BLK, NE), 1)
    v1 = jnp.max(logits, axis=1, keepdims=True)
    i1 = jnp.min(jnp.where(logits == v1, iota8, NE), axis=1, keepdims=True)
    masked = jnp.where(iota8 == i1, -jnp.inf, logits)
    v2 = jnp.max(masked, axis=1, keepdims=True)
    i2 = jnp.min(jnp.where(masked == v2, iota8, NE), axis=1, keepdims=True)
    gv1 = 1.0 / (1.0 + jnp.exp(v2 - v1))
    gv2 = 1.0 - gv1
    topi_ref[...] = jnp.concatenate([i1, i2], axis=1)
    gd = jnp.where(iota8 == i1, gv1, 0.0) + jnp.where(iota8 == i2, gv2, 0.0)
    gd = jnp.where(valid, gd, 0.0)

    @pl.when(i == 0)
    def _():
        imp_ref[...] = jnp.zeros_like(imp_ref)
        st2_ref[...] = jnp.zeros_like(st2_ref)

    imp_ref[...] += jnp.sum(gd, axis=0, keepdims=True)

    z = jnp.concatenate([h, aggx, agge], axis=1)
    big = jnp.dot(z, wall_ref[...], preferred_element_type=jnp.float32)
    big = jnp.maximum(big + bbf_ref[...], 0.0)
    big = big.astype(jnp.bfloat16).astype(jnp.float32)
    gdt = gd.astype(jnp.bfloat16).astype(jnp.float32)
    acc = jnp.zeros((BLK, Dout), jnp.float32)
    for e in range(NE):
        acc = acc + gdt[:, e:e + 1] * big[:, e * Dout:(e + 1) * Dout]
    if no == 1:
        out_refs[0][...] = acc
    else:
        for q in range(no):
            out_refs[q][...] = acc[:, q * 128:(q + 1) * 128]

    st2_ref[...] += jnp.concatenate(
        [jnp.sum(acc, axis=0, keepdims=True),
         jnp.sum(acc * acc, axis=0, keepdims=True)], axis=0)

    @pl.when(i == NG - 1)
    def _():
        imp = imp_ref[...]
        mi = jnp.sum(imp) / NE
        vi = jnp.sum((imp - mi) ** 2) / NE
        loss_ref[...] = (vi / (mi * mi + 1e-10)).reshape(1, 1)


def _expert_layer(xs, aggs, agge, stats, g, b, Wg, Wall, bbf,
                  Din, Dout, no=1):
    Z = 2 * Din + DE
    nx = len(xs)
    nh = len(aggs)
    body = functools.partial(_expert_body, Din, Dout, nx, nh, no)

    def _part_spec():
        return pl.BlockSpec((NC, BLK, 128), lambda i: (0, i, 0))

    if no == 1:
        o_specs = [pl.BlockSpec((BLK, Dout), lambda i: (i, 0))]
        o_shapes = [jax.ShapeDtypeStruct((N, Dout), jnp.float32)]
    else:
        o_specs = [pl.BlockSpec((BLK, 128), lambda i: (i, 0))] * no
        o_shapes = [jax.ShapeDtypeStruct((N, 128), jnp.float32)] * no

    return pl.pallas_call(
        body,
        grid=(NG,),
        in_specs=[
            *[pl.BlockSpec((BLK, Din // nx), lambda i: (i, 0))
              for _ in xs],
            *[_part_spec() for _ in aggs],
            _part_spec(),
            pl.BlockSpec((2, Din), lambda i: (0, 0)),
            pl.BlockSpec((1, Din), lambda i: (0, 0)),
            pl.BlockSpec((1, Din), lambda i: (0, 0)),
            pl.BlockSpec((Din, NE), lambda i: (0, 0)),
            pl.BlockSpec((Z, NE * Dout), lambda i: (0, 0)),
            pl.BlockSpec((1, NE * Dout), lambda i: (0, 0)),
        ],
        out_specs=[
            *o_specs,
            pl.BlockSpec((BLK, 2), lambda i: (i, 0)),
            pl.BlockSpec((1, NE), lambda i: (0, 0)),
            pl.BlockSpec((1, 1), lambda i: (0, 0)),
            pl.BlockSpec((2, Dout), lambda i: (0, 0)),
        ],
        out_shape=[
            *o_shapes,
            jax.ShapeDtypeStruct((N, 2), jnp.int32),
            jax.ShapeDtypeStruct((1, NE), jnp.float32),
            jax.ShapeDtypeStruct((1, 1), jnp.float32),
            jax.ShapeDtypeStruct((2, Dout), jnp.float32),
        ],
    )(*xs, *aggs, agge, stats, g, b, Wg, Wall, bbf)


def _pack_weights(Ws, Wm, We, bb):
    W8 = jnp.concatenate([Ws, Wm, We], axis=1)
    Z = W8.shape[1]
    Dout = W8.shape[2]
    Wall = jnp.transpose(W8, (1, 0, 2)).reshape(Z, NE * Dout)
    return Wall, bb.reshape(1, NE * Dout)


def kernel(x, edge_index, edge_attr, g1, b1, g2, b2,
           Wg1, Ws1, Wm1, We1, bb1, Wg2, Ws2, Wm2, We2, bb2):
    src3d = edge_index[0].reshape(NBLK, 1, EB)
    dst3d = edge_index[1].reshape(NBLK, 1, EB)
    ea128 = jnp.concatenate(
        [edge_attr, jnp.ones((E, 1), jnp.float32),
         jnp.zeros((E, 128 - DE - 1), jnp.float32)], axis=1)
    z128 = jnp.zeros((WCHUNK, 128), jnp.float32)

    aggE = _make_scpass(False)(ea128, src3d, dst3d, z128)
    aggX = _make_scpass(True)(x, src3d, dst3d, z128)
    stats1 = _stats(x)

    Wall1, bbf1 = _pack_weights(Ws1, Wm1, We1, bb1)
    h2a, h2b, e1, imp1, l1, st2 = _expert_layer(
        [x], [aggX], aggE, stats1,
        g1.reshape(1, IN), b1.reshape(1, IN), Wg1, Wall1, bbf1,
        IN, HID, no=2)
    del imp1

    aggB0 = _make_scpass(True)(h2a, src3d, dst3d, z128)
    aggB1 = _make_scpass(True)(h2b, src3d, dst3d, z128)

    Wall2, bbf2 = _pack_weights(Ws2, Wm2, We2, bb2)
    out, e2, imp2, l2, _ = _expert_layer(
        [h2a, h2b], [aggB0, aggB1], aggE, st2,
        g2.reshape(1, HID), b2.reshape(1, HID), Wg2, Wall2, bbf2, HID, OUT)
    del imp2

    losses = jnp.stack([l1[0, 0], l2[0, 0]])
    return out, e1, e2, losses
